# Initial kernel scaffold; baseline (speedup 1.0000x reference)
import jax
import jax.numpy as jnp
from jax.experimental import pallas as pl


def kernel(Z, edge_index, W1, b1, W2, b2):
    raise NotImplementedError("write your pallas kernel here")


# speedup vs baseline: 4.5707x; 4.5707x over previous
import functools

import jax
import jax.numpy as jnp
from jax import lax
from jax.experimental import pallas as pl
from jax.experimental.pallas import tpu as pltpu
from jax.experimental.pallas import tpu_sc as plsc

EMBED = 128
HID = 64
NODES = 10000
EDGES = 320000

_info = plsc.get_sparse_core_info()
_NC = _info.num_cores
_NS = _info.num_subcores
NW = _NC * _NS
EPW = EDGES // NW
CHUNK = 80
NCHUNK = EPW // CHUNK


def _tables_body(z_ref, w1_ref, b1_ref, hs_ref, hd_ref):
    z = z_ref[...]
    w1 = w1_ref[...]
    hs = lax.dot_general(z, w1[:, :EMBED], (((1,), (1,)), ((), ())),
                         preferred_element_type=jnp.float32)
    hd = lax.dot_general(z, w1[:, EMBED:], (((1,), (1,)), ((), ())),
                         preferred_element_type=jnp.float32)
    hs_ref[...] = hs
    hd_ref[...] = hd + b1_ref[...]


_tables = pl.pallas_call(
    _tables_body,
    out_shape=(
        jax.ShapeDtypeStruct((NODES, HID), jnp.float32),
        jax.ShapeDtypeStruct((NODES, HID), jnp.float32),
    ),
)


def _sc_body(hs_hbm, hd_hbm, src_hbm, dst_hbm, w2e_hbm, out_hbm,
             idx_s, idx_d, rows_s, rows_d, out_v, w2v, tbuf, sem):
    wid = lax.axis_index("s") * _NC + lax.axis_index("c")
    base = wid * EPW
    pltpu.sync_copy(w2e_hbm, w2v)
    pltpu.sync_copy(src_hbm.at[pl.ds(base, EPW)], idx_s)
    pltpu.sync_copy(dst_hbm.at[pl.ds(base, EPW)], idx_d)
    w0 = w2v[0:16]
    w1 = w2v[16:32]
    w2 = w2v[32:48]
    w3 = w2v[48:64]
    bvec = w2v[64:80]

    col_iota = lax.iota(jnp.int32, 16) * 16

    def chunk_body(i, _):
        off = i * CHUNK
        pltpu.async_copy(hs_hbm.at[idx_s.at[pl.ds(off, CHUNK)]], rows_s, sem).wait()
        pltpu.async_copy(hd_hbm.at[idx_d.at[pl.ds(off, CHUNK)]], rows_d, sem).wait()

        def group_body(g, _):
            gbase = g * 16
            for lane in range(16):
                e = gbase + lane
                t0 = jnp.maximum(rows_s[e, 0:16] + rows_d[e, 0:16], 0.0)
                t1 = jnp.maximum(rows_s[e, 16:32] + rows_d[e, 16:32], 0.0)
                t2 = jnp.maximum(rows_s[e, 32:48] + rows_d[e, 32:48], 0.0)
                t3 = jnp.maximum(rows_s[e, 48:64] + rows_d[e, 48:64], 0.0)
                acc = t0 * w0 + t1 * w1 + t2 * w2 + t3 * w3
                plsc.store_scatter(tbuf, [col_iota + lane], acc)
            res = bvec
            for j in range(16):
                res = res + tbuf[pl.ds(j * 16, 16)]
            out_v[pl.ds(off + gbase, 16)] = res
            return 0

        lax.fori_loop(0, CHUNK // 16, group_body, 0)
        return 0

    lax.fori_loop(0, NCHUNK, chunk_body, 0)
    pltpu.sync_copy(out_v, out_hbm.at[pl.ds(base, EPW)])


_sc_kernel = functools.partial(
    pl.kernel,
    out_type=jax.ShapeDtypeStruct((EDGES,), jnp.float32),
    mesh=plsc.VectorSubcoreMesh(core_axis_name="c", subcore_axis_name="s"),
    compiler_params=pltpu.CompilerParams(needs_layout_passes=False,
                                         use_tc_tiling_on_sc=False),
    scratch_types=[
        pltpu.VMEM((EPW,), jnp.int32),
        pltpu.VMEM((EPW,), jnp.int32),
        pltpu.VMEM((CHUNK, HID), jnp.float32),
        pltpu.VMEM((CHUNK, HID), jnp.float32),
        pltpu.VMEM((EPW,), jnp.float32),
        pltpu.VMEM((80,), jnp.float32),
        pltpu.VMEM((256,), jnp.float32),
        pltpu.SemaphoreType.DMA,
    ],
)(_sc_body)


def kernel(Z, edge_index, W1, b1, W2, b2):
    hs, hd = _tables(Z, W1, b1.reshape(1, HID).astype(jnp.float32))
    ei = edge_index.astype(jnp.int32)
    w2e = jnp.concatenate([W2.reshape(HID),
                           jnp.broadcast_to(b2.reshape(1), (16,))])
    return _sc_kernel(hs, hd, ei[0], ei[1], w2e)
